# baseline (speedup 1.0000x reference)
import functools

import jax
import jax.numpy as jnp
from jax import lax
from jax.experimental import pallas as pl
from jax.experimental.pallas import tpu as pltpu
from jax.experimental.pallas import tpu_sc as plsc

BATCH = 4096
SEQ = 200
HIDDEN = 64
NUM_LABELS = 128
PADW = 128

NUM_CORES = 2
NUM_SUBCORES = 16
NUM_WORKERS = NUM_CORES * NUM_SUBCORES
LANES = 16

ROWS_PER_WORKER = BATCH // NUM_WORKERS
CHUNK0 = 128
CHUNK1 = SEQ - CHUNK0


def _pool_body(idx_hbm, table_hbm, out_hbm, idx_v, rows_v, out_v, sem):
    wid = lax.axis_index("s") * NUM_CORES + lax.axis_index("c")
    base_row = wid * ROWS_PER_WORKER

    pltpu.sync_copy(
        idx_hbm.at[pl.ds(base_row * SEQ, ROWS_PER_WORKER * SEQ)], idx_v
    )

    def row_body(r, carry):
        off0 = pl.multiple_of(r * SEQ, 8)
        off1 = pl.multiple_of(r * SEQ + CHUNK0, 8)
        c0 = pltpu.async_copy(
            table_hbm.at[idx_v.at[pl.ds(off0, CHUNK0)]],
            rows_v.at[pl.ds(0, CHUNK0)],
            sem,
        )
        c1 = pltpu.async_copy(
            table_hbm.at[idx_v.at[pl.ds(off1, CHUNK1)]],
            rows_v.at[pl.ds(CHUNK0, CHUNK1)],
            sem,
        )
        c0.wait()
        c1.wait()

        def s_body(s, acc):
            return tuple(
                acc[j] + rows_v[s, pl.ds(j * LANES, LANES)] for j in range(4)
            )

        zero = jnp.zeros((LANES,), jnp.float32)
        acc = lax.fori_loop(0, SEQ, s_body, (zero, zero, zero, zero))
        for j in range(4):
            out_v[r, pl.ds(j * LANES, LANES)] = acc[j]
        return carry

    lax.fori_loop(0, ROWS_PER_WORKER, row_body, 0)
    pltpu.sync_copy(out_v, out_hbm.at[pl.ds(base_row, ROWS_PER_WORKER)])


_pool = pl.kernel(
    _pool_body,
    out_type=jax.ShapeDtypeStruct((BATCH, HIDDEN), jnp.float32),
    mesh=plsc.VectorSubcoreMesh(
        core_axis_name="c", subcore_axis_name="s", num_cores=NUM_CORES
    ),
    scratch_types=[
        pltpu.VMEM((ROWS_PER_WORKER * SEQ,), jnp.int32),
        pltpu.VMEM((SEQ, PADW), jnp.float32),
        pltpu.VMEM((ROWS_PER_WORKER, HIDDEN), jnp.float32),
        pltpu.SemaphoreType.DMA,
    ],
    compiler_params=pltpu.CompilerParams(use_tc_tiling_on_sc=True),
)


def _head_body(x_ref, w_ref, b_ref, o_ref):
    x = x_ref[...] * (1.0 / SEQ)
    logits = (
        lax.dot_general(
            x,
            w_ref[...],
            (((1,), (1,)), ((), ())),
            preferred_element_type=jnp.float32,
        )
        + b_ref[...]
    )
    m = jnp.max(logits, axis=1, keepdims=True)
    e = jnp.exp(logits - m)
    s = jnp.sum(e, axis=1, keepdims=True)
    o_ref[...] = (logits - m) - jnp.log(s)


def _head(pooled, W, b2d):
    return pl.pallas_call(
        _head_body,
        grid=(1,),
        in_specs=[
            pl.BlockSpec((BATCH, HIDDEN), lambda i: (0, 0)),
            pl.BlockSpec((NUM_LABELS, HIDDEN), lambda i: (0, 0)),
            pl.BlockSpec((1, NUM_LABELS), lambda i: (0, 0)),
        ],
        out_specs=pl.BlockSpec((BATCH, NUM_LABELS), lambda i: (0, 0)),
        out_shape=jax.ShapeDtypeStruct((BATCH, NUM_LABELS), jnp.float32),
    )(pooled, W, b2d)


@jax.jit
def kernel(one_hot_sentence, emb_table, W, b):
    idx = one_hot_sentence.reshape(-1).astype(jnp.int32)
    tab128 = jnp.pad(emb_table, ((0, 0), (0, PADW - HIDDEN)))
    pooled = _pool(idx, tab128)
    return _head(pooled, W, b.reshape(1, NUM_LABELS))


# speedup vs baseline: 1.0071x; 1.0071x over previous
import functools

import jax
import jax.numpy as jnp
from jax import lax
from jax.experimental import pallas as pl
from jax.experimental.pallas import tpu as pltpu
from jax.experimental.pallas import tpu_sc as plsc

VOCAB = 1000000
BATCH = 4096
SEQ = 200
HIDDEN = 64
NUM_LABELS = 128

NUM_CORES = 2
NUM_SUBCORES = 16
NUM_WORKERS = NUM_CORES * NUM_SUBCORES
LANES = 16

ROWS_PER_WORKER = BATCH // NUM_WORKERS
CHUNK0 = 128
CHUNK1 = SEQ - CHUNK0


BI = 2048


def _project_body(t_ref, w_ref, b_ref, o_ref):
    o_ref[...] = (
        lax.dot_general(
            t_ref[...],
            w_ref[...],
            (((0,), (1,)), ((), ())),
            preferred_element_type=jnp.float32,
        )
        + b_ref[...]
    )


def _project(tableT, W, b2d):
    return pl.pallas_call(
        _project_body,
        grid=(pl.cdiv(VOCAB, BI),),
        in_specs=[
            pl.BlockSpec((HIDDEN, BI), lambda i: (0, i)),
            pl.BlockSpec((NUM_LABELS, HIDDEN), lambda i: (0, 0)),
            pl.BlockSpec((1, NUM_LABELS), lambda i: (0, 0)),
        ],
        out_specs=pl.BlockSpec((BI, NUM_LABELS), lambda i: (i, 0)),
        out_shape=jax.ShapeDtypeStruct((VOCAB, NUM_LABELS), jnp.float32),
    )(tableT, W, b2d)


def _pool_body(idx_hbm, table_hbm, out_hbm, idx_v, rows_v, out_v, sem):
    wid = lax.axis_index("s") * NUM_CORES + lax.axis_index("c")
    base_row = wid * ROWS_PER_WORKER

    pltpu.sync_copy(
        idx_hbm.at[pl.ds(base_row * SEQ, ROWS_PER_WORKER * SEQ)], idx_v
    )

    def row_body(r, carry):
        off0 = pl.multiple_of(r * SEQ, 8)
        off1 = pl.multiple_of(r * SEQ + CHUNK0, 8)
        c0 = pltpu.async_copy(
            table_hbm.at[idx_v.at[pl.ds(off0, CHUNK0)]],
            rows_v.at[pl.ds(0, CHUNK0)],
            sem,
        )
        c1 = pltpu.async_copy(
            table_hbm.at[idx_v.at[pl.ds(off1, CHUNK1)]],
            rows_v.at[pl.ds(CHUNK0, CHUNK1)],
            sem,
        )
        c0.wait()
        c1.wait()

        def s_body(s, acc):
            return tuple(
                acc[j] + rows_v[s, pl.ds(j * LANES, LANES)] for j in range(8)
            )

        zero = jnp.zeros((LANES,), jnp.float32)
        acc = lax.fori_loop(0, SEQ, s_body, (zero,) * 8)
        for j in range(8):
            out_v[r, pl.ds(j * LANES, LANES)] = acc[j]
        return carry

    lax.fori_loop(0, ROWS_PER_WORKER, row_body, 0)
    pltpu.sync_copy(out_v, out_hbm.at[pl.ds(base_row, ROWS_PER_WORKER)])


_pool = pl.kernel(
    _pool_body,
    out_type=jax.ShapeDtypeStruct((BATCH, NUM_LABELS), jnp.float32),
    mesh=plsc.VectorSubcoreMesh(
        core_axis_name="c", subcore_axis_name="s", num_cores=NUM_CORES
    ),
    scratch_types=[
        pltpu.VMEM((ROWS_PER_WORKER * SEQ,), jnp.int32),
        pltpu.VMEM((SEQ, NUM_LABELS), jnp.float32),
        pltpu.VMEM((ROWS_PER_WORKER, NUM_LABELS), jnp.float32),
        pltpu.SemaphoreType.DMA,
    ],
    compiler_params=pltpu.CompilerParams(use_tc_tiling_on_sc=True),
)


def _head_body(x_ref, o_ref):
    logits = x_ref[...] * (1.0 / SEQ)
    m = jnp.max(logits, axis=1, keepdims=True)
    e = jnp.exp(logits - m)
    s = jnp.sum(e, axis=1, keepdims=True)
    o_ref[...] = (logits - m) - jnp.log(s)


def _head(summed):
    return pl.pallas_call(
        _head_body,
        grid=(4,),
        in_specs=[pl.BlockSpec((BATCH // 4, NUM_LABELS), lambda i: (i, 0))],
        out_specs=pl.BlockSpec((BATCH // 4, NUM_LABELS), lambda i: (i, 0)),
        out_shape=jax.ShapeDtypeStruct((BATCH, NUM_LABELS), jnp.float32),
    )(summed)


@jax.jit
def kernel(one_hot_sentence, emb_table, W, b):
    idx = one_hot_sentence.reshape(-1).astype(jnp.int32)
    proj = _project(emb_table.T, W, b.reshape(1, NUM_LABELS))
    summed = _pool(idx, proj)
    return _head(summed)


# speedup vs baseline: 1.7854x; 1.7729x over previous
import functools

import jax
import jax.numpy as jnp
from jax import lax
from jax.experimental import pallas as pl
from jax.experimental.pallas import tpu as pltpu
from jax.experimental.pallas import tpu_sc as plsc

VOCAB = 1000000
BATCH = 4096
SEQ = 200
HIDDEN = 64
NUM_LABELS = 128

NUM_CORES = 2
NUM_SUBCORES = 16
NUM_WORKERS = NUM_CORES * NUM_SUBCORES
LANES = 16

ROWS_PER_WORKER = BATCH // NUM_WORKERS
CHUNK0 = 128
CHUNK1 = SEQ - CHUNK0


BI = 8192


def _project_body(t_ref, w_ref, b_ref, o_ref):
    o_ref[...] = (
        lax.dot_general(
            t_ref[...],
            w_ref[...],
            (((0,), (1,)), ((), ())),
            preferred_element_type=jnp.float32,
        )
        + b_ref[...]
    )


def _project(tableT, W, b2d):
    return pl.pallas_call(
        _project_body,
        grid=(pl.cdiv(VOCAB, BI),),
        in_specs=[
            pl.BlockSpec((HIDDEN, BI), lambda i: (0, i)),
            pl.BlockSpec((NUM_LABELS, HIDDEN), lambda i: (0, 0)),
            pl.BlockSpec((1, NUM_LABELS), lambda i: (0, 0)),
        ],
        out_specs=pl.BlockSpec((BI, NUM_LABELS), lambda i: (i, 0)),
        out_shape=jax.ShapeDtypeStruct((VOCAB, NUM_LABELS), jnp.float32),
    )(tableT, W, b2d)


def _pool_body(idx_hbm, table_hbm, out_hbm, idx_v, rows_v, out_v, sem0, sem1):
    wid = lax.axis_index("s") * NUM_CORES + lax.axis_index("c")
    base_row = wid * ROWS_PER_WORKER

    pltpu.sync_copy(
        idx_hbm.at[pl.ds(base_row * SEQ, ROWS_PER_WORKER * SEQ)], idx_v
    )

    sems = (sem0, sem1)

    def gather_descs(r, buf):
        off0 = pl.multiple_of(r * SEQ, 8)
        off1 = pl.multiple_of(r * SEQ + CHUNK0, 8)
        return (
            (
                table_hbm.at[idx_v.at[pl.ds(off0, CHUNK0)]],
                rows_v.at[buf].at[pl.ds(0, CHUNK0)],
                sems[buf],
            ),
            (
                table_hbm.at[idx_v.at[pl.ds(off1, CHUNK1)]],
                rows_v.at[buf].at[pl.ds(CHUNK0, CHUNK1)],
                sems[buf],
            ),
        )

    def start(r, buf):
        for desc in gather_descs(r, buf):
            pltpu.async_copy(*desc)

    def wait(r, buf):
        for desc in gather_descs(r, buf):
            pltpu.make_async_copy(*desc).wait()

    def accum(r, buf):
        def s_body(s, acc):
            return tuple(
                acc[j] + rows_v[buf, s, pl.ds(j * LANES, LANES)]
                for j in range(8)
            )

        zero = jnp.zeros((LANES,), jnp.float32)
        acc = lax.fori_loop(0, SEQ, s_body, (zero,) * 8)
        for j in range(8):
            out_v[r, pl.ds(j * LANES, LANES)] = acc[j]

    start(0, 0)

    def pair_body(k, carry):
        r0 = k * 2
        start(r0 + 1, 1)
        wait(r0, 0)
        accum(r0, 0)

        @pl.when(k < ROWS_PER_WORKER // 2 - 1)
        def _():
            start(r0 + 2, 0)

        wait(r0 + 1, 1)
        accum(r0 + 1, 1)
        return carry

    lax.fori_loop(0, ROWS_PER_WORKER // 2, pair_body, 0)
    pltpu.sync_copy(out_v, out_hbm.at[pl.ds(base_row, ROWS_PER_WORKER)])


_pool = pl.kernel(
    _pool_body,
    out_type=jax.ShapeDtypeStruct((BATCH, NUM_LABELS), jnp.float32),
    mesh=plsc.VectorSubcoreMesh(
        core_axis_name="c", subcore_axis_name="s", num_cores=NUM_CORES
    ),
    scratch_types=[
        pltpu.VMEM((ROWS_PER_WORKER * SEQ,), jnp.int32),
        pltpu.VMEM((2, SEQ, NUM_LABELS), jnp.float32),
        pltpu.VMEM((ROWS_PER_WORKER, NUM_LABELS), jnp.float32),
        pltpu.SemaphoreType.DMA,
        pltpu.SemaphoreType.DMA,
    ],
    compiler_params=pltpu.CompilerParams(use_tc_tiling_on_sc=True),
)


def _head_body(x_ref, o_ref):
    logits = x_ref[...] * (1.0 / SEQ)
    m = jnp.max(logits, axis=1, keepdims=True)
    e = jnp.exp(logits - m)
    s = jnp.sum(e, axis=1, keepdims=True)
    o_ref[...] = (logits - m) - jnp.log(s)


def _head(summed):
    return pl.pallas_call(
        _head_body,
        grid=(4,),
        in_specs=[pl.BlockSpec((BATCH // 4, NUM_LABELS), lambda i: (i, 0))],
        out_specs=pl.BlockSpec((BATCH // 4, NUM_LABELS), lambda i: (i, 0)),
        out_shape=jax.ShapeDtypeStruct((BATCH, NUM_LABELS), jnp.float32),
    )(summed)


@jax.jit
def kernel(one_hot_sentence, emb_table, W, b):
    idx = one_hot_sentence.reshape(-1).astype(jnp.int32)
    proj = _project(emb_table.T, W, b.reshape(1, NUM_LABELS))
    summed = _pool(idx, proj)
    return _head(summed)
